# Initial kernel scaffold; baseline (speedup 1.0000x reference)
import jax
import jax.numpy as jnp
from jax.experimental import pallas as pl


def kernel(pred_scores, box_id_to_token_id, token_id_maps, src_idx):
    raise NotImplementedError("write your pallas kernel here")


# speedup vs baseline: 5.7250x; 5.7250x over previous
import functools

import jax
import jax.numpy as jnp
from jax import lax
from jax.experimental import pallas as pl
from jax.experimental.pallas import tpu as pltpu
from jax.experimental.pallas import tpu_sc as plsc

B = 8
M = 131072
K = 1024
G = 3
HW = 128
STRIDE = M + 8

NC, NS = 2, 16
B_PER_CORE = B // NC
TPB = NS // B_PER_CORE
KPT = K // TPB
NKC = KPT // 128
NCHUNK = G * KPT // 128

_MESH = plsc.VectorSubcoreMesh(
    core_axis_name="c", subcore_axis_name="s", num_cores=NC, num_subcores=NS
)

_SC_SCRATCH = (
    [pltpu.VMEM((KPT,), jnp.int32)]
    + [pltpu.VMEM((128,), jnp.int32) for _ in range(2 * NKC)]
    + [pltpu.VMEM((128,), jnp.int32) for _ in range(5 * NCHUNK)]
    + [pltpu.VMEM((128,), jnp.int32) for _ in range(NCHUNK)]
    + [pltpu.VMEM((128,), jnp.float32) for _ in range(NCHUNK)]
    + [pltpu.VMEM((16,), jnp.float32)]
)


def _sc_body(tmap_hbm, rc_hbm, src_hbm, pred_hbm, out_hbm, mark_hbm, *scr):
    src_v = scr[0]
    bidx = scr[1:1 + NKC]
    rcv = scr[3:3 + NKC]
    o = 1 + 2 * NKC
    midx = scr[o:o + NCHUNK]
    tval = scr[o + 6:o + 6 + NCHUNK]
    pbuf = scr[o + 12:o + 12 + NCHUNK]
    ebuf = scr[o + 18:o + 18 + NCHUNK]
    wbuf = scr[o + 24:o + 24 + NCHUNK]
    xibuf = scr[o + 30:o + 30 + NCHUNK]
    xv = scr[o + 36:o + 36 + NCHUNK]
    accv = scr[o + 42]

    c = lax.axis_index("c")
    s = lax.axis_index("s")
    b = c * B_PER_CORE + s // TPB
    tb = s % TPB
    wid = c * NS + s

    pltpu.sync_copy(src_hbm.at[pl.ds(b * K + tb * KPT, KPT)], src_v)

    iot = lax.broadcasted_iota(jnp.int32, (16,), 0)
    for q in range(NKC):
        for i in range(8):
            kv = src_v[pl.ds(q * 128 + i * 16, 16)]
            bidx[q][pl.ds(i * 16, 16)] = (kv * 10923) >> 16
    for q in range(NKC):
        pltpu.sync_copy(rc_hbm.at[bidx[q]], rcv[q])
    for q in range(NKC):
        for i in range(8):
            rc = rcv[q][pl.ds(i * 16, 16)]
            for g in range(G):
                e_off = g * KPT + q * 128 + i * 16
                j, off = divmod(e_off, 128)
                midx[j][pl.ds(off, 16)] = rc + (g * B + b) * (HW * HW)

    for j in range(NCHUNK):
        pltpu.sync_copy(tmap_hbm.at[midx[j]], tval[j])

    ebase = (b * TPB + tb) * (G * KPT)
    for j in range(NCHUNK):
        for i in range(8):
            sl = pl.ds(i * 16, 16)
            t = tval[j][sl]
            valid = t >= 0
            pbuf[j][sl] = jnp.where(valid, t, M) + b * STRIDE
            ebuf[j][sl] = ebase + j * 128 + i * 16 + iot
            xibuf[j][sl] = jnp.where(valid, t, 0) + b * M

    for j in range(NCHUNK):
        pltpu.sync_copy(ebuf[j], mark_hbm.at[pbuf[j]])
    plsc.subcore_barrier()

    for j in range(NCHUNK):
        pltpu.sync_copy(mark_hbm.at[pbuf[j]], wbuf[j])
        pltpu.sync_copy(pred_hbm.at[xibuf[j]], xv[j])

    acc = jnp.zeros((16,), jnp.float32)
    for j in range(NCHUNK):
        for i in range(8):
            sl = pl.ds(i * 16, 16)
            keep = (wbuf[j][sl] == ebuf[j][sl]) & (tval[j][sl] >= 0)
            acc = acc + jnp.where(keep, xv[j][sl], 0.0)
    accv[...] = acc
    pltpu.sync_copy(accv, out_hbm.at[wid])


_sc_sparse = functools.partial(
    pl.kernel,
    out_type=[
        jax.ShapeDtypeStruct((NC * NS, 16), jnp.float32),
        jax.ShapeDtypeStruct((B * STRIDE,), jnp.int32),
    ],
    mesh=_MESH,
    scratch_types=_SC_SCRATCH,
)(_sc_body)


def _dense_body(x_ref, p_ref, o_ref):
    x = x_ref[...]
    dense = jnp.sum(jnp.maximum(x, 0.0) + jnp.log1p(jnp.exp(-jnp.abs(x))))
    o_ref[0, 0] = (dense - jnp.sum(p_ref[...])) * (1.0 / (B * M))


_dense = pl.pallas_call(
    _dense_body,
    out_shape=jax.ShapeDtypeStruct((1, 1), jnp.float32),
    out_specs=pl.BlockSpec(memory_space=pltpu.SMEM),
)


def kernel(pred_scores, box_id_to_token_id, token_id_maps, src_idx):
    n_box = box_id_to_token_id.shape[0]
    rc = box_id_to_token_id[:, 0] * HW + box_id_to_token_id[:, 1]
    rc = jnp.concatenate([rc, jnp.zeros((K - n_box,), jnp.int32)])
    partials, _ = _sc_sparse(
        token_id_maps.reshape(-1),
        rc,
        src_idx.reshape(-1),
        pred_scores.reshape(-1),
    )
    return _dense(pred_scores, partials)[0, 0]


# speedup vs baseline: 5.9379x; 1.0372x over previous
import functools

import jax
import jax.numpy as jnp
from jax import lax
from jax.experimental import pallas as pl
from jax.experimental.pallas import tpu as pltpu
from jax.experimental.pallas import tpu_sc as plsc

B = 8
M = 131072
K = 1024
G = 3
HW = 128
STRIDE = M + 8

NC, NS = 2, 16
B_PER_CORE = B // NC
TPB = NS // B_PER_CORE
KPT = K // TPB
NKC = KPT // 128
NCHUNK = G * KPT // 128

_MESH = plsc.VectorSubcoreMesh(
    core_axis_name="c", subcore_axis_name="s", num_cores=NC, num_subcores=NS
)

_SC_SCRATCH = (
    [pltpu.VMEM((KPT,), jnp.int32)]
    + [pltpu.VMEM((128,), jnp.int32) for _ in range(2 * NKC)]
    + [pltpu.VMEM((128,), jnp.int32) for _ in range(5 * NCHUNK)]
    + [pltpu.VMEM((128,), jnp.int32) for _ in range(NCHUNK)]
    + [pltpu.VMEM((128,), jnp.float32) for _ in range(NCHUNK)]
    + [pltpu.VMEM((16,), jnp.float32)]
    + [pltpu.SemaphoreType.DMA, pltpu.SemaphoreType.DMA]
)


def _sc_body(tmap_hbm, rc_hbm, src_hbm, pred_hbm, out_hbm, mark_hbm, *scr):
    src_v = scr[0]
    bidx = scr[1:1 + NKC]
    rcv = scr[3:3 + NKC]
    o = 1 + 2 * NKC
    midx = scr[o:o + NCHUNK]
    tval = scr[o + 6:o + 6 + NCHUNK]
    pbuf = scr[o + 12:o + 12 + NCHUNK]
    ebuf = scr[o + 18:o + 18 + NCHUNK]
    wbuf = scr[o + 24:o + 24 + NCHUNK]
    xibuf = scr[o + 30:o + 30 + NCHUNK]
    xv = scr[o + 36:o + 36 + NCHUNK]
    accv = scr[o + 42]
    sem_a, sem_b = scr[o + 43], scr[o + 44]

    c = lax.axis_index("c")
    s = lax.axis_index("s")
    b = c * B_PER_CORE + s // TPB
    tb = s % TPB
    wid = c * NS + s

    pltpu.sync_copy(src_hbm.at[pl.ds(b * K + tb * KPT, KPT)], src_v)

    iot = lax.broadcasted_iota(jnp.int32, (16,), 0)
    for q in range(NKC):
        for i in range(8):
            kv = src_v[pl.ds(q * 128 + i * 16, 16)]
            bidx[q][pl.ds(i * 16, 16)] = (kv * 10923) >> 16
    rc_d = [pltpu.async_copy(rc_hbm.at[bidx[q]], rcv[q], sem_a) for q in range(NKC)]
    for d in rc_d:
        d.wait()
    for q in range(NKC):
        for i in range(8):
            rc = rcv[q][pl.ds(i * 16, 16)]
            for g in range(G):
                e_off = g * KPT + q * 128 + i * 16
                j, off = divmod(e_off, 128)
                midx[j][pl.ds(off, 16)] = rc + (g * B + b) * (HW * HW)

    tm_d = [pltpu.async_copy(tmap_hbm.at[midx[j]], tval[j], sem_a)
            for j in range(NCHUNK)]
    for d in tm_d:
        d.wait()

    ebase = (b * TPB + tb) * (G * KPT)
    for j in range(NCHUNK):
        for i in range(8):
            sl = pl.ds(i * 16, 16)
            t = tval[j][sl]
            valid = t >= 0
            pbuf[j][sl] = jnp.where(valid, t, M) + b * STRIDE
            ebuf[j][sl] = ebase + j * 128 + i * 16 + iot
            xibuf[j][sl] = jnp.where(valid, t, 0) + b * M

    sc_d = [pltpu.async_copy(ebuf[j], mark_hbm.at[pbuf[j]], sem_a)
            for j in range(NCHUNK)]
    xg_d = [pltpu.async_copy(pred_hbm.at[xibuf[j]], xv[j], sem_b)
            for j in range(NCHUNK)]
    for d in sc_d:
        d.wait()
    plsc.subcore_barrier()

    wg_d = [pltpu.async_copy(mark_hbm.at[pbuf[j]], wbuf[j], sem_a)
            for j in range(NCHUNK)]
    for d in wg_d:
        d.wait()
    for d in xg_d:
        d.wait()

    acc = jnp.zeros((16,), jnp.float32)
    for j in range(NCHUNK):
        for i in range(8):
            sl = pl.ds(i * 16, 16)
            keep = (wbuf[j][sl] == ebuf[j][sl]) & (tval[j][sl] >= 0)
            acc = acc + jnp.where(keep, xv[j][sl], 0.0)
    accv[...] = acc
    pltpu.sync_copy(accv, out_hbm.at[wid])


_sc_sparse = functools.partial(
    pl.kernel,
    out_type=[
        jax.ShapeDtypeStruct((NC * NS, 16), jnp.float32),
        jax.ShapeDtypeStruct((B * STRIDE,), jnp.int32),
    ],
    mesh=_MESH,
    scratch_types=_SC_SCRATCH,
)(_sc_body)


def _dense_body(x_ref, p_ref, o_ref):
    x = x_ref[...]
    dense = jnp.sum(jnp.maximum(x, 0.0) + jnp.log1p(jnp.exp(-jnp.abs(x))))
    o_ref[0, 0] = (dense - jnp.sum(p_ref[...])) * (1.0 / (B * M))


_dense = pl.pallas_call(
    _dense_body,
    out_shape=jax.ShapeDtypeStruct((1, 1), jnp.float32),
    out_specs=pl.BlockSpec(memory_space=pltpu.SMEM),
)


def kernel(pred_scores, box_id_to_token_id, token_id_maps, src_idx):
    n_box = box_id_to_token_id.shape[0]
    rc = box_id_to_token_id[:, 0] * HW + box_id_to_token_id[:, 1]
    rc = jnp.concatenate([rc, jnp.zeros((K - n_box,), jnp.int32)])
    partials, _ = _sc_sparse(
        token_id_maps.reshape(-1),
        rc,
        src_idx.reshape(-1),
        pred_scores.reshape(-1),
    )
    return _dense(pred_scores, partials)[0, 0]


# speedup vs baseline: 6.2840x; 1.0583x over previous
import functools

import jax
import jax.numpy as jnp
from jax import lax
from jax.experimental import pallas as pl
from jax.experimental.pallas import tpu as pltpu
from jax.experimental.pallas import tpu_sc as plsc

B = 8
M = 131072
K = 1024
G = 3
HW = 128
STRIDE = M + 8

NC, NS = 2, 16
B_PER_CORE = B // NC
TPB = NS // B_PER_CORE
KPT = K // TPB
NKC = KPT // 128
NCHUNK = G * KPT // 128

_MESH = plsc.VectorSubcoreMesh(
    core_axis_name="c", subcore_axis_name="s", num_cores=NC, num_subcores=NS
)

_SC_SCRATCH = (
    [pltpu.VMEM((KPT,), jnp.int32)]
    + [pltpu.VMEM((128,), jnp.int32) for _ in range(2 * NKC)]
    + [pltpu.VMEM((128,), jnp.int32) for _ in range(5 * NCHUNK)]
    + [pltpu.VMEM((128,), jnp.int32) for _ in range(NCHUNK)]
    + [pltpu.VMEM((128,), jnp.float32) for _ in range(NCHUNK)]
    + [pltpu.VMEM((16,), jnp.float32)]
    + [pltpu.SemaphoreType.DMA, pltpu.SemaphoreType.DMA]
)


def _sc_body(tmap_hbm, rc_hbm, src_hbm, pred_hbm, out_hbm, mark_hbm, *scr):
    src_v = scr[0]
    bidx = scr[1:1 + NKC]
    rcv = scr[3:3 + NKC]
    o = 1 + 2 * NKC
    midx = scr[o:o + NCHUNK]
    tval = scr[o + 6:o + 6 + NCHUNK]
    pbuf = scr[o + 12:o + 12 + NCHUNK]
    ebuf = scr[o + 18:o + 18 + NCHUNK]
    wbuf = scr[o + 24:o + 24 + NCHUNK]
    xibuf = scr[o + 30:o + 30 + NCHUNK]
    xv = scr[o + 36:o + 36 + NCHUNK]
    accv = scr[o + 42]
    sem_a, sem_b = scr[o + 43], scr[o + 44]

    c = lax.axis_index("c")
    s = lax.axis_index("s")
    b = c * B_PER_CORE + s // TPB
    tb = s % TPB
    wid = c * NS + s

    with jax.named_scope("p0_src"):
        pltpu.sync_copy(src_hbm.at[pl.ds(b * K + tb * KPT, KPT)], src_v)

    iot = lax.broadcasted_iota(jnp.int32, (16,), 0)
    for q in range(NKC):
        for i in range(8):
            kv = src_v[pl.ds(q * 128 + i * 16, 16)]
            bidx[q][pl.ds(i * 16, 16)] = (kv * 10923) >> 16
    with jax.named_scope("p1_rc"):
        rc_d = [pltpu.async_copy(rc_hbm.at[bidx[q]], rcv[q], sem_a) for q in range(NKC)]
        for d in rc_d:
            d.wait()
    for q in range(NKC):
        for i in range(8):
            rc = rcv[q][pl.ds(i * 16, 16)]
            for g in range(G):
                e_off = g * KPT + q * 128 + i * 16
                j, off = divmod(e_off, 128)
                midx[j][pl.ds(off, 16)] = rc + (g * B + b) * (HW * HW)

    with jax.named_scope("p2_tmap"):
        tm_d = [pltpu.async_copy(tmap_hbm.at[midx[j]], tval[j], sem_a)
                for j in range(NCHUNK)]
        for d in tm_d:
            d.wait()

    ebase = (b * TPB + tb) * (G * KPT)
    for j in range(NCHUNK):
        for i in range(8):
            sl = pl.ds(i * 16, 16)
            t = tval[j][sl]
            valid = t >= 0
            pbuf[j][sl] = jnp.where(valid, t, M) + b * STRIDE
            ebuf[j][sl] = ebase + j * 128 + i * 16 + iot
            xibuf[j][sl] = jnp.where(valid, t, 0) + b * M

    with jax.named_scope("p4_scatter"):
        sc_d = [pltpu.async_copy(ebuf[j], mark_hbm.at[pbuf[j]], sem_a)
                for j in range(NCHUNK)]
        xg_d = [pltpu.async_copy(pred_hbm.at[xibuf[j]], xv[j], sem_b)
                for j in range(NCHUNK)]
        for d in sc_d:
            d.wait()
    with jax.named_scope("p4b_barrier"):
        plsc.subcore_barrier()

    with jax.named_scope("p5_wback"):
        wg_d = [pltpu.async_copy(mark_hbm.at[pbuf[j]], wbuf[j], sem_a)
                for j in range(NCHUNK)]
        for d in wg_d:
            d.wait()
        for d in xg_d:
            d.wait()

    acc = jnp.zeros((16,), jnp.float32)
    for j in range(NCHUNK):
        for i in range(8):
            sl = pl.ds(i * 16, 16)
            keep = (wbuf[j][sl] == ebuf[j][sl]) & (tval[j][sl] >= 0)
            acc = acc + jnp.where(keep, xv[j][sl], 0.0)
    accv[...] = acc
    pltpu.sync_copy(accv, out_hbm.at[wid])


_sc_sparse = functools.partial(
    pl.kernel,
    out_type=[
        jax.ShapeDtypeStruct((NC * NS, 16), jnp.float32),
        jax.ShapeDtypeStruct((B * STRIDE,), jnp.int32),
    ],
    mesh=_MESH,
    scratch_types=_SC_SCRATCH,
)(_sc_body)


_DN = 16
_DBLK = M // _DN


def _dense_body(x_ref, o_ref):
    x = x_ref[...]
    blk = jnp.sum(jnp.maximum(x, 0.0) + jnp.log1p(jnp.exp(-jnp.abs(x))))
    i = pl.program_id(0)

    @pl.when(i == 0)
    def _():
        o_ref[0, 0] = 0.0

    o_ref[0, 0] += blk


_dense = pl.pallas_call(
    _dense_body,
    grid=(_DN,),
    in_specs=[pl.BlockSpec((B, _DBLK), lambda i: (0, i))],
    out_shape=jax.ShapeDtypeStruct((1, 1), jnp.float32),
    out_specs=pl.BlockSpec(memory_space=pltpu.SMEM),
)


def kernel(pred_scores, box_id_to_token_id, token_id_maps, src_idx):
    n_box = box_id_to_token_id.shape[0]
    rc = box_id_to_token_id[:, 0] * HW + box_id_to_token_id[:, 1]
    rc = jnp.concatenate([rc, jnp.zeros((K - n_box,), jnp.int32)])
    partials, _ = _sc_sparse(
        token_id_maps.reshape(-1),
        rc,
        src_idx.reshape(-1),
        pred_scores.reshape(-1),
    )
    dense = _dense(pred_scores)[0, 0]
    return (dense - jnp.sum(partials)) * (1.0 / (B * M))


# speedup vs baseline: 6.3043x; 1.0032x over previous
import functools

import jax
import jax.numpy as jnp
from jax import lax
from jax.experimental import pallas as pl
from jax.experimental.pallas import tpu as pltpu
from jax.experimental.pallas import tpu_sc as plsc

B = 8
M = 131072
K = 1024
G = 3
HW = 128
STRIDE = M + 8

NC, NS = 2, 16
B_PER_CORE = B // NC
TPB = NS // B_PER_CORE
KPT = K // TPB
NKC = KPT // 128
NCHUNK = G * KPT // 128

_MESH = plsc.VectorSubcoreMesh(
    core_axis_name="c", subcore_axis_name="s", num_cores=NC, num_subcores=NS
)

_SC_SCRATCH = (
    [pltpu.VMEM((KPT,), jnp.int32)]
    + [pltpu.VMEM((128,), jnp.int32) for _ in range(2 * NKC)]
    + [pltpu.VMEM((128,), jnp.int32) for _ in range(5 * NCHUNK)]
    + [pltpu.VMEM((128,), jnp.int32) for _ in range(NCHUNK)]
    + [pltpu.VMEM((128,), jnp.float32) for _ in range(NCHUNK)]
    + [pltpu.VMEM((16,), jnp.float32)]
    + [pltpu.SemaphoreType.DMA, pltpu.SemaphoreType.DMA]
)


def _sc_body(tmap_hbm, rc_hbm, src_hbm, pred_hbm, out_hbm, mark_hbm, *scr):
    src_v = scr[0]
    bidx = scr[1:1 + NKC]
    rcv = scr[3:3 + NKC]
    o = 1 + 2 * NKC
    midx = scr[o:o + NCHUNK]
    tval = scr[o + 6:o + 6 + NCHUNK]
    pbuf = scr[o + 12:o + 12 + NCHUNK]
    ebuf = scr[o + 18:o + 18 + NCHUNK]
    wbuf = scr[o + 24:o + 24 + NCHUNK]
    xibuf = scr[o + 30:o + 30 + NCHUNK]
    xv = scr[o + 36:o + 36 + NCHUNK]
    accv = scr[o + 42]
    sem_a, sem_b = scr[o + 43], scr[o + 44]

    c = lax.axis_index("c")
    s = lax.axis_index("s")
    b = c * B_PER_CORE + s // TPB
    tb = s % TPB
    wid = c * NS + s

    with jax.named_scope("p0_src"):
        pltpu.sync_copy(src_hbm.at[pl.ds(b * K + tb * KPT, KPT)], src_v)

    iot = lax.broadcasted_iota(jnp.int32, (16,), 0)
    with jax.named_scope("p1a_bidx"):
        for q in range(NKC):
            for i in range(8):
                kv = src_v[pl.ds(q * 128 + i * 16, 16)]
                bidx[q][pl.ds(i * 16, 16)] = (kv * 10923) >> 16
    with jax.named_scope("p1_rc"):
        rc_d = [pltpu.async_copy(rc_hbm.at[bidx[q]], rcv[q], sem_a) for q in range(NKC)]
        for d in rc_d:
            d.wait()
    with jax.named_scope("p1c_midx"):
        for q in range(NKC):
            for i in range(8):
                rc = rcv[q][pl.ds(i * 16, 16)]
                for g in range(G):
                    e_off = g * KPT + q * 128 + i * 16
                    j, off = divmod(e_off, 128)
                    midx[j][pl.ds(off, 16)] = rc + (g * B + b) * (HW * HW)

    with jax.named_scope("p2_tmap"):
        tm_d = [pltpu.async_copy(tmap_hbm.at[midx[j]], tval[j], sem_a)
                for j in range(NCHUNK)]
        for d in tm_d:
            d.wait()

    ebase = (b * TPB + tb) * (G * KPT)
    with jax.named_scope("p3_mark"):
        for j in range(NCHUNK):
            for i in range(8):
                sl = pl.ds(i * 16, 16)
                t = tval[j][sl]
                valid = t >= 0
                pbuf[j][sl] = jnp.where(valid, t, M) + b * STRIDE
                ebuf[j][sl] = ebase + j * 128 + i * 16 + iot
                xibuf[j][sl] = jnp.where(valid, t, 0) + b * M

    with jax.named_scope("p4_scatter"):
        sc_d = [pltpu.async_copy(ebuf[j], mark_hbm.at[pbuf[j]], sem_a)
                for j in range(NCHUNK)]
        xg_d = [pltpu.async_copy(pred_hbm.at[xibuf[j]], xv[j], sem_b)
                for j in range(NCHUNK)]
        for d in sc_d:
            d.wait()
    with jax.named_scope("p4b_barrier"):
        plsc.subcore_barrier()

    with jax.named_scope("p5_wback"):
        wg_d = [pltpu.async_copy(mark_hbm.at[pbuf[j]], wbuf[j], sem_a)
                for j in range(NCHUNK)]
        for d in wg_d:
            d.wait()
        for d in xg_d:
            d.wait()

    with jax.named_scope("p6_acc"):
        acc = jnp.zeros((16,), jnp.float32)
        for j in range(NCHUNK):
            for i in range(8):
                sl = pl.ds(i * 16, 16)
                keep = (wbuf[j][sl] == ebuf[j][sl]) & (tval[j][sl] >= 0)
                acc = acc + jnp.where(keep, xv[j][sl], 0.0)
        accv[...] = acc
    with jax.named_scope("p7_out"):
        pltpu.sync_copy(accv, out_hbm.at[wid])


_sc_sparse = functools.partial(
    pl.kernel,
    out_type=[
        jax.ShapeDtypeStruct((NC * NS, 16), jnp.float32),
        jax.ShapeDtypeStruct((B * STRIDE,), jnp.int32),
    ],
    mesh=_MESH,
    scratch_types=_SC_SCRATCH,
)(_sc_body)


_DN = 16
_DBLK = M // _DN


def _dense_body(x_ref, o_ref):
    x = x_ref[...]
    blk = jnp.sum(jnp.maximum(x, 0.0) + jnp.log1p(jnp.exp(-jnp.abs(x))))
    i = pl.program_id(0)

    @pl.when(i == 0)
    def _():
        o_ref[0, 0] = 0.0

    o_ref[0, 0] += blk


_dense = pl.pallas_call(
    _dense_body,
    grid=(_DN,),
    in_specs=[pl.BlockSpec((B, _DBLK), lambda i: (0, i))],
    out_shape=jax.ShapeDtypeStruct((1, 1), jnp.float32),
    out_specs=pl.BlockSpec(memory_space=pltpu.SMEM),
)


def kernel(pred_scores, box_id_to_token_id, token_id_maps, src_idx):
    n_box = box_id_to_token_id.shape[0]
    rc = box_id_to_token_id[:, 0] * HW + box_id_to_token_id[:, 1]
    rc = jnp.concatenate([rc, jnp.zeros((K - n_box,), jnp.int32)])
    partials, _ = _sc_sparse(
        token_id_maps.reshape(-1),
        rc,
        src_idx.reshape(-1),
        pred_scores.reshape(-1),
    )
    dense = _dense(pred_scores)[0, 0]
    return (dense - jnp.sum(partials)) * (1.0 / (B * M))


# speedup vs baseline: 6.8675x; 1.0893x over previous
import functools

import jax
import jax.numpy as jnp
from jax import lax
from jax.experimental import pallas as pl
from jax.experimental.pallas import tpu as pltpu
from jax.experimental.pallas import tpu_sc as plsc

B = 8
M = 131072
K = 1024
G = 3
HW = 128
STRIDE = M + 8

NC, NS = 2, 16
NW = NC * NS
B_PER_CORE = B // NC
TPB = NS // B_PER_CORE
KPT = K // TPB
NKC = KPT // 128
NCHUNK = G * KPT // 128

_MESH = plsc.VectorSubcoreMesh(
    core_axis_name="c", subcore_axis_name="s", num_cores=NC, num_subcores=NS
)

_SC_SCRATCH = [
    pltpu.VMEM((KPT,), jnp.int32),
    pltpu.VMEM((NKC, 128), jnp.int32),
    pltpu.VMEM((NKC, 128), jnp.int32),
    pltpu.VMEM((NCHUNK, 128), jnp.int32),
    pltpu.VMEM((NCHUNK, 128), jnp.int32),
    pltpu.VMEM((NCHUNK, 128), jnp.int32),
    pltpu.VMEM((NCHUNK, 128), jnp.int32),
    pltpu.VMEM((NCHUNK, 128), jnp.int32),
    pltpu.VMEM((NCHUNK, 128), jnp.int32),
    pltpu.VMEM((NCHUNK, 128), jnp.float32),
    pltpu.VMEM((16,), jnp.float32),
    pltpu.SemaphoreType.DMA,
    pltpu.SemaphoreType.DMA,
]


def _sc_body(tmap_hbm, rcrep_hbm, src_hbm, pred_hbm, out_hbm, mark_hbm,
             src_v, bidx, rcv, midx, tval, pbuf, ebuf, wbuf, xibuf, xv,
             accv, sem_a, sem_b):
    c = lax.axis_index("c")
    s = lax.axis_index("s")
    b = c * B_PER_CORE + s // TPB
    tb = s % TPB
    wid = c * NS + s

    with jax.named_scope("p0_src"):
        pltpu.sync_copy(src_hbm.at[pl.ds(b * K + tb * KPT, KPT)], src_v)

    iot = lax.broadcasted_iota(jnp.int32, (16,), 0)

    rbase = wid * K
    with jax.named_scope("p1a_bidx"):
        for q in range(NKC):
            def _f1(i, _, q=q):
                kv = src_v[pl.ds(q * 128 + i * 16, 16)]
                bidx[q, pl.ds(i * 16, 16)] = ((kv * 10923) >> 16) + rbase
                return 0
            lax.fori_loop(0, 8, _f1, 0)

    with jax.named_scope("p1_rc"):
        rc_d = [pltpu.async_copy(rcrep_hbm.at[bidx.at[q]], rcv.at[q], sem_a)
                for q in range(NKC)]
        for d in rc_d:
            d.wait()

    with jax.named_scope("p1c_midx"):
        for g in range(G):
            gb = (g * B + b) * (HW * HW)
            for q in range(NKC):
                def _f2(i, _, g=g, q=q, gb=gb):
                    midx[g * NKC + q, pl.ds(i * 16, 16)] = (
                        rcv[q, pl.ds(i * 16, 16)] + gb)
                    return 0
                lax.fori_loop(0, 8, _f2, 0)

    with jax.named_scope("p2_tmap"):
        tm_d = [pltpu.async_copy(tmap_hbm.at[midx.at[j]], tval.at[j], sem_a)
                for j in range(NCHUNK)]
        for d in tm_d:
            d.wait()

    ebase = (b * TPB + tb) * (G * KPT)
    with jax.named_scope("p3_mark"):
        for j in range(NCHUNK):
            def _f3(i, _, j=j):
                sl = pl.ds(i * 16, 16)
                t = tval[j, sl]
                valid = t >= 0
                pbuf[j, sl] = jnp.where(valid, t, M) + b * STRIDE
                ebuf[j, sl] = ebase + j * 128 + i * 16 + iot
                xibuf[j, sl] = jnp.where(valid, t, 0) + b * M
                return 0
            lax.fori_loop(0, 8, _f3, 0)

    with jax.named_scope("p4_scatter"):
        sc_d = [pltpu.async_copy(ebuf.at[j], mark_hbm.at[pbuf.at[j]], sem_a)
                for j in range(NCHUNK)]
        xg_d = [pltpu.async_copy(pred_hbm.at[xibuf.at[j]], xv.at[j], sem_b)
                for j in range(NCHUNK)]
        for d in sc_d:
            d.wait()
    with jax.named_scope("p4b_barrier"):
        plsc.subcore_barrier()

    with jax.named_scope("p5_wback"):
        wg_d = [pltpu.async_copy(mark_hbm.at[pbuf.at[j]], wbuf.at[j], sem_a)
                for j in range(NCHUNK)]
        for d in wg_d:
            d.wait()
        for d in xg_d:
            d.wait()

    with jax.named_scope("p6_acc"):
        acc = jnp.zeros((16,), jnp.float32)
        for j in range(NCHUNK):
            def _f4(i, a, j=j):
                sl = pl.ds(i * 16, 16)
                keep = (wbuf[j, sl] == ebuf[j, sl]) & (tval[j, sl] >= 0)
                return a + jnp.where(keep, xv[j, sl], 0.0)
            acc = lax.fori_loop(0, 8, _f4, acc)
        accv[...] = acc
    with jax.named_scope("p7_out"):
        pltpu.sync_copy(accv, out_hbm.at[wid])


_sc_sparse = functools.partial(
    pl.kernel,
    out_type=[
        jax.ShapeDtypeStruct((NW, 16), jnp.float32),
        jax.ShapeDtypeStruct((B * STRIDE,), jnp.int32),
    ],
    mesh=_MESH,
    scratch_types=_SC_SCRATCH,
)(_sc_body)


_DN = 16
_DBLK = M // _DN


def _dense_body(x_ref, o_ref):
    x = x_ref[...]
    blk = jnp.sum(jnp.maximum(x, 0.0) + jnp.log1p(jnp.exp(-jnp.abs(x))))
    i = pl.program_id(0)

    @pl.when(i == 0)
    def _():
        o_ref[0, 0] = 0.0

    o_ref[0, 0] += blk


_dense = pl.pallas_call(
    _dense_body,
    grid=(_DN,),
    in_specs=[pl.BlockSpec((B, _DBLK), lambda i: (0, i))],
    out_shape=jax.ShapeDtypeStruct((1, 1), jnp.float32),
    out_specs=pl.BlockSpec(memory_space=pltpu.SMEM),
)


def kernel(pred_scores, box_id_to_token_id, token_id_maps, src_idx):
    n_box = box_id_to_token_id.shape[0]
    rc = box_id_to_token_id[:, 0] * HW + box_id_to_token_id[:, 1]
    rc = jnp.concatenate([rc, jnp.zeros((K - n_box,), jnp.int32)])
    rc_rep = jnp.tile(rc, NW)
    partials, _ = _sc_sparse(
        token_id_maps.reshape(-1),
        rc_rep,
        src_idx.reshape(-1),
        pred_scores.reshape(-1),
    )
    dense = _dense(pred_scores)[0, 0]
    return (dense - jnp.sum(partials)) * (1.0 / (B * M))


# speedup vs baseline: 6.8994x; 1.0046x over previous
import functools

import jax
import jax.numpy as jnp
from jax import lax
from jax.experimental import pallas as pl
from jax.experimental.pallas import tpu as pltpu
from jax.experimental.pallas import tpu_sc as plsc

B = 8
M = 131072
K = 1024
G = 3
HW = 128
STRIDE = M + 8

NC, NS = 2, 16
NW = NC * NS
B_PER_CORE = B // NC
TPB = NS // B_PER_CORE
KPT = K // TPB
NKC = KPT // 128
NCHUNK = G * KPT // 128

_MESH = plsc.VectorSubcoreMesh(
    core_axis_name="c", subcore_axis_name="s", num_cores=NC, num_subcores=NS
)

_SC_SCRATCH = [
    pltpu.VMEM((KPT,), jnp.int32),
    pltpu.VMEM((NKC, 128), jnp.int32),
    pltpu.VMEM((NKC, 128), jnp.int32),
    pltpu.VMEM((NCHUNK, 128), jnp.int32),
    pltpu.VMEM((NCHUNK, 128), jnp.int32),
    pltpu.VMEM((NCHUNK, 128), jnp.int32),
    pltpu.VMEM((NCHUNK, 128), jnp.int32),
    pltpu.VMEM((NCHUNK, 128), jnp.int32),
    pltpu.VMEM((NCHUNK, 128), jnp.float32),
    pltpu.VMEM((16,), jnp.float32),
    pltpu.SemaphoreType.DMA,
    pltpu.SemaphoreType.DMA,
]


def _sc_body(tmap_hbm, rcrep_hbm, src_hbm, pred_hbm, out_hbm, mark_hbm,
             src_v, bidx, rcv, tval, pbuf, ebuf, wbuf, xibuf, xv,
             accv, sem_a, sem_b):
    c = lax.axis_index("c")
    s = lax.axis_index("s")
    b = c * B_PER_CORE + s // TPB
    tb = s % TPB
    wid = c * NS + s

    with jax.named_scope("p0_src"):
        pltpu.sync_copy(src_hbm.at[pl.ds(b * K + tb * KPT, KPT)], src_v)

    iot = lax.broadcasted_iota(jnp.int32, (16,), 0)

    rbase = wid * K
    with jax.named_scope("p1a_bidx"):
        def _f1(i, _):
            kv = src_v[pl.ds(i * 16, 16)]
            bidx[i >> 3, pl.ds((i & 7) * 16, 16)] = ((kv * 10923) >> 16) + rbase
            return 0
        lax.fori_loop(0, 16, _f1, 0)

    with jax.named_scope("p1_rc"):
        rc_d = [pltpu.async_copy(rcrep_hbm.at[bidx.at[q]], rcv.at[q], sem_a)
                for q in range(NKC)]
        for d in rc_d:
            d.wait()

    with jax.named_scope("p2_tmap"):
        tm_d = [
            pltpu.async_copy(
                tmap_hbm.at[pl.ds(((j >> 1) * B + b) * (HW * HW), HW * HW)]
                .at[rcv.at[j & 1]],
                tval.at[j], sem_a)
            for j in range(NCHUNK)
        ]
        for d in tm_d:
            d.wait()

    ebase = (b * TPB + tb) * (G * KPT)
    with jax.named_scope("p3_mark"):
        def _f3(i, _):
            j = i >> 3
            sl = pl.ds((i & 7) * 16, 16)
            t = tval[j, sl]
            valid = t >= 0
            pbuf[j, sl] = jnp.where(valid, t, M) + b * STRIDE
            ebuf[j, sl] = ebase + i * 16 + iot
            xibuf[j, sl] = jnp.where(valid, t, 0) + b * M
            return 0
        lax.fori_loop(0, 48, _f3, 0)

    with jax.named_scope("p4_scatter"):
        sc_d = [pltpu.async_copy(ebuf.at[j], mark_hbm.at[pbuf.at[j]], sem_a)
                for j in range(NCHUNK)]
        xg_d = [pltpu.async_copy(pred_hbm.at[xibuf.at[j]], xv.at[j], sem_b)
                for j in range(NCHUNK)]
        for d in sc_d:
            d.wait()
    with jax.named_scope("p4b_barrier"):
        plsc.subcore_barrier()

    with jax.named_scope("p5_wback"):
        wg_d = [pltpu.async_copy(mark_hbm.at[pbuf.at[j]], wbuf.at[j], sem_a)
                for j in range(NCHUNK)]
        for d in wg_d:
            d.wait()
        for d in xg_d:
            d.wait()

    with jax.named_scope("p6_acc"):
        def _f4(i, a):
            j = i >> 3
            sl = pl.ds((i & 7) * 16, 16)
            keep = (wbuf[j, sl] == ebuf[j, sl]) & (tval[j, sl] >= 0)
            return a + jnp.where(keep, xv[j, sl], 0.0)
        accv[...] = lax.fori_loop(0, 48, _f4, jnp.zeros((16,), jnp.float32))
    with jax.named_scope("p7_out"):
        pltpu.sync_copy(accv, out_hbm.at[wid])


_sc_sparse = functools.partial(
    pl.kernel,
    out_type=[
        jax.ShapeDtypeStruct((NW, 16), jnp.float32),
        jax.ShapeDtypeStruct((B * STRIDE,), jnp.int32),
    ],
    mesh=_MESH,
    scratch_types=_SC_SCRATCH,
)(_sc_body)


_DN = 16
_DBLK = M // _DN


def _dense_body(x_ref, o_ref):
    x = x_ref[...]
    blk = jnp.sum(jnp.maximum(x, 0.0) + jnp.log1p(jnp.exp(-jnp.abs(x))))
    i = pl.program_id(0)

    @pl.when(i == 0)
    def _():
        o_ref[0, 0] = 0.0

    o_ref[0, 0] += blk


_dense = pl.pallas_call(
    _dense_body,
    grid=(_DN,),
    in_specs=[pl.BlockSpec((B, _DBLK), lambda i: (0, i))],
    out_shape=jax.ShapeDtypeStruct((1, 1), jnp.float32),
    out_specs=pl.BlockSpec(memory_space=pltpu.SMEM),
)


def kernel(pred_scores, box_id_to_token_id, token_id_maps, src_idx):
    n_box = box_id_to_token_id.shape[0]
    rc = box_id_to_token_id[:, 0] * HW + box_id_to_token_id[:, 1]
    rc = jnp.concatenate([rc, jnp.zeros((K - n_box,), jnp.int32)])
    rc_rep = jnp.tile(rc, NW)
    partials, _ = _sc_sparse(
        token_id_maps.reshape(-1),
        rc_rep,
        src_idx.reshape(-1),
        pred_scores.reshape(-1),
    )
    dense = _dense(pred_scores)[0, 0]
    return (dense - jnp.sum(partials)) * (1.0 / (B * M))


# speedup vs baseline: 10.6590x; 1.5449x over previous
import functools

import jax
import jax.numpy as jnp
from jax import lax
from jax.experimental import pallas as pl
from jax.experimental.pallas import tpu as pltpu
from jax.experimental.pallas import tpu_sc as plsc

B = 8
M = 131072
K = 1024
G = 3
HW = 128
STRIDE = M + 8

NC, NS = 2, 16
NW = NC * NS
B_PER_CORE = B // NC
TPB = NS // B_PER_CORE
KPT = K // TPB
NKC = KPT // 128
NCHUNK = G * KPT // 128

_MESH = plsc.VectorSubcoreMesh(
    core_axis_name="c", subcore_axis_name="s", num_cores=NC, num_subcores=NS
)

_SC_SCRATCH = [
    pltpu.VMEM((KPT,), jnp.int32),
    pltpu.VMEM((NKC, 128), jnp.int32),
    pltpu.VMEM((NKC, 128), jnp.int32),
    pltpu.VMEM((NCHUNK, 128), jnp.int32),
    pltpu.VMEM((NCHUNK, 128), jnp.int32),
    pltpu.VMEM((NCHUNK, 128), jnp.int32),
    pltpu.VMEM((NCHUNK, 128), jnp.int32),
    pltpu.VMEM((NCHUNK, 128), jnp.int32),
    pltpu.VMEM((NCHUNK, 128), jnp.float32),
    pltpu.VMEM((16,), jnp.float32),
    pltpu.VMEM_SHARED((B_PER_CORE * STRIDE,), jnp.int32),
    pltpu.SemaphoreType.DMA,
    pltpu.SemaphoreType.DMA,
]


def _sc_body(tmap_hbm, rcrep_hbm, src_hbm, pred_hbm, out_hbm,
             src_v, bidx, rcv, tval, pbuf, ebuf, wbuf, xibuf, xv,
             accv, mark_sp, sem_a, sem_b):
    c = lax.axis_index("c")
    s = lax.axis_index("s")
    bc = s // TPB
    b = c * B_PER_CORE + bc
    tb = s % TPB
    wid = c * NS + s

    with jax.named_scope("p0_src"):
        pltpu.sync_copy(src_hbm.at[pl.ds(b * K + tb * KPT, KPT)], src_v)

    iot = lax.broadcasted_iota(jnp.int32, (16,), 0)

    rbase = wid * K
    with jax.named_scope("p1a_bidx"):
        def _f1(i, _):
            kv = src_v[pl.ds(i * 16, 16)]
            bidx[i >> 3, pl.ds((i & 7) * 16, 16)] = ((kv * 10923) >> 16) + rbase
            return 0
        lax.fori_loop(0, 16, _f1, 0)

    with jax.named_scope("p1_rc"):
        rc_d = [pltpu.async_copy(rcrep_hbm.at[bidx.at[q]], rcv.at[q], sem_a)
                for q in range(NKC)]
        for d in rc_d:
            d.wait()

    with jax.named_scope("p2_tmap"):
        tm_d = [
            pltpu.async_copy(
                tmap_hbm.at[pl.ds(((j >> 1) * B + b) * (HW * HW), HW * HW)]
                .at[rcv.at[j & 1]],
                tval.at[j], sem_a)
            for j in range(NCHUNK)
        ]
        for d in tm_d:
            d.wait()

    ebase = (b * TPB + tb) * (G * KPT)
    with jax.named_scope("p3_mark"):
        def _f3(i, _):
            j = i >> 3
            sl = pl.ds((i & 7) * 16, 16)
            t = tval[j, sl]
            valid = t >= 0
            pbuf[j, sl] = jnp.where(valid, t, M) + bc * STRIDE
            ebuf[j, sl] = ebase + i * 16 + iot
            xibuf[j, sl] = jnp.where(valid, t, 0) + b * M
            return 0
        lax.fori_loop(0, 48, _f3, 0)

    with jax.named_scope("p4_scatter"):
        sc_d = [pltpu.async_copy(ebuf.at[j], mark_sp.at[pbuf.at[j]], sem_a)
                for j in range(NCHUNK)]
        xg_d = [pltpu.async_copy(pred_hbm.at[xibuf.at[j]], xv.at[j], sem_b)
                for j in range(NCHUNK)]
        for d in sc_d:
            d.wait()
    with jax.named_scope("p4b_barrier"):
        plsc.subcore_barrier()

    with jax.named_scope("p5_wback"):
        wg_d = [pltpu.async_copy(mark_sp.at[pbuf.at[j]], wbuf.at[j], sem_a)
                for j in range(NCHUNK)]
        for d in wg_d:
            d.wait()
        for d in xg_d:
            d.wait()

    with jax.named_scope("p6_acc"):
        def _f4(i, a):
            j = i >> 3
            sl = pl.ds((i & 7) * 16, 16)
            keep = (wbuf[j, sl] == ebuf[j, sl]) & (tval[j, sl] >= 0)
            return a + jnp.where(keep, xv[j, sl], 0.0)
        accv[...] = lax.fori_loop(0, 48, _f4, jnp.zeros((16,), jnp.float32))
    with jax.named_scope("p7_out"):
        pltpu.sync_copy(accv, out_hbm.at[wid])


_sc_sparse = functools.partial(
    pl.kernel,
    out_type=jax.ShapeDtypeStruct((NW, 16), jnp.float32),
    mesh=_MESH,
    scratch_types=_SC_SCRATCH,
)(_sc_body)


_DN = 16
_DBLK = M // _DN


def _dense_body(x_ref, o_ref):
    x = x_ref[...]
    blk = jnp.sum(jnp.maximum(x, 0.0) + jnp.log1p(jnp.exp(-jnp.abs(x))))
    i = pl.program_id(0)

    @pl.when(i == 0)
    def _():
        o_ref[0, 0] = 0.0

    o_ref[0, 0] += blk


_dense = pl.pallas_call(
    _dense_body,
    grid=(_DN,),
    in_specs=[pl.BlockSpec((B, _DBLK), lambda i: (0, i))],
    out_shape=jax.ShapeDtypeStruct((1, 1), jnp.float32),
    out_specs=pl.BlockSpec(memory_space=pltpu.SMEM),
)


def _combine_body(d_ref, p_ref, o_ref):
    o_ref[0, 0] = (d_ref[0, 0] - jnp.sum(p_ref[...])) * (1.0 / (B * M))


_combine = pl.pallas_call(
    _combine_body,
    in_specs=[pl.BlockSpec(memory_space=pltpu.SMEM), pl.BlockSpec()],
    out_shape=jax.ShapeDtypeStruct((1, 1), jnp.float32),
    out_specs=pl.BlockSpec(memory_space=pltpu.SMEM),
)


def kernel(pred_scores, box_id_to_token_id, token_id_maps, src_idx):
    n_box = box_id_to_token_id.shape[0]
    rc = box_id_to_token_id[:, 0] * HW + box_id_to_token_id[:, 1]
    rc = jnp.concatenate([rc, jnp.zeros((K - n_box,), jnp.int32)])
    rc_rep = jnp.tile(rc, NW)
    partials = _sc_sparse(
        token_id_maps.reshape(-1),
        rc_rep,
        src_idx.reshape(-1),
        pred_scores.reshape(-1),
    )
    dense = _dense(pred_scores)
    return _combine(dense, partials)[0, 0]
